# baseline (speedup 1.0000x reference)
import jax
import jax.numpy as jnp
from jax import lax
from jax.experimental import pallas as pl
from jax.experimental.pallas import tpu as pltpu
from jax.experimental.pallas import tpu_sc as plsc

B, S = 16384, 50
D = 64
N_ROWS = B * S
L = 128
NC, NS = 2, 16
NW = NC * NS
B_PER_W = B // NW
CB = 16
CROWS = CB * S
NBUF = 2
N_CHUNKS = B_PER_W // CB
N_GROUPS = N_CHUNKS // NBUF
NLANES = 16
assert B_PER_W % (CB * NBUF) == 0


def _gather_body(idx_hbm, table_hbm, out_hbm, idx_v, flat_v, rows_v,
                 isems, gsems, osems):
    wid = lax.axis_index("s") * NC + lax.axis_index("c")
    bbase = wid * B_PER_W
    lane = lax.iota(jnp.int32, NLANES)

    def issue_idx(j, b):
        pltpu.async_copy(
            idx_hbm.at[pl.ds(bbase + j * CB, CB)], idx_v.at[b], isems[b])

    def wait_idx(b):
        pltpu.make_async_copy(
            idx_hbm.at[pl.ds(bbase, CB)], idx_v.at[b], isems[b]).wait()

    def compact(b):
        dst = flat_v.at[b]
        for r in range(CB):
            for v in range(0, S, NLANES):
                x = idx_v[b, r, pl.ds(v, NLANES)]
                offs = lane + (r * S + v)
                if v + NLANES <= S:
                    plsc.store_scatter(dst, [offs], x)
                else:
                    plsc.store_scatter(dst, [offs], x, mask=lane < (S - v))

    def issue_gather(b):
        pltpu.async_copy(table_hbm.at[flat_v.at[b]], rows_v.at[b], gsems[b])

    def wait_gather(b):
        pltpu.make_async_copy(
            table_hbm.at[flat_v.at[b]], rows_v.at[b], gsems[b]).wait()

    def issue_out(j, b):
        pltpu.async_copy(
            rows_v.at[b],
            out_hbm.at[pl.ds((bbase + j * CB) * S, CROWS)], osems[b])

    def wait_out(b):
        pltpu.make_async_copy(
            rows_v.at[b], out_hbm.at[pl.ds(bbase * S, CROWS)],
            osems[b]).wait()

    def start_chunk(b):
        wait_idx(b)
        compact(b)
        issue_gather(b)

    def finalize(k, b, last):
        wait_gather(b)
        issue_out(k, b)
        if not last:
            issue_idx(jnp.minimum(k + NBUF, N_CHUNKS - 1), b)

    for b in range(NBUF):
        issue_idx(b, b)
    for b in range(NBUF):
        start_chunk(b)
        if b > 0:
            finalize(b - 1, b - 1, last=False)

    @pl.loop(1, N_GROUPS)
    def _group(g):
        j0 = g * NBUF
        for b in range(NBUF):
            j = j0 + b
            wait_out(b)
            start_chunk(b)
            pb = (b - 1) % NBUF
            finalize(j - 1, pb, last=False)

    last_b = (N_CHUNKS - 1) % NBUF
    finalize(N_CHUNKS - 1, last_b, last=True)
    for b in range(NBUF):
        wait_out(b)
    for b in range(NBUF):
        if b != last_b:
            wait_idx(b)


def kernel(word_idx, emb):
    idxp = jnp.pad(word_idx, ((0, 0), (0, L - S)))
    mesh = plsc.VectorSubcoreMesh(core_axis_name="c", subcore_axis_name="s")
    f = pl.kernel(
        _gather_body,
        out_type=jax.ShapeDtypeStruct((N_ROWS, D), jnp.float32),
        mesh=mesh,
        scratch_types=[
            pltpu.VMEM((NBUF, CB, L), jnp.int32),
            pltpu.VMEM((NBUF, CROWS), jnp.int32),
            pltpu.VMEM((NBUF, CROWS, D), jnp.float32),
            [pltpu.SemaphoreType.DMA] * NBUF,
            [pltpu.SemaphoreType.DMA] * NBUF,
            [pltpu.SemaphoreType.DMA] * NBUF,
        ],
        compiler_params=pltpu.CompilerParams(
            use_tc_tiling_on_sc=False, needs_layout_passes=False),
    )
    out = f(idxp, emb)
    return out.reshape(B, S, D)


# speedup vs baseline: 1.0052x; 1.0052x over previous
import jax
import jax.numpy as jnp
from jax import lax
from jax.experimental import pallas as pl
from jax.experimental.pallas import tpu as pltpu
from jax.experimental.pallas import tpu_sc as plsc

B, S = 16384, 50
D = 64
N_ROWS = B * S
NC, NS = 2, 16
NW = NC * NS
B_PER_W = B // NW
CB = 16
CROWS = CB * S
NBUF = 2
N_CHUNKS = B_PER_W // CB
N_GROUPS = N_CHUNKS // NBUF
NLANES = 16
assert B_PER_W % (CB * NBUF) == 0
assert CB == NLANES


def _gather_body(idx_hbm, table_hbm, out_hbm, idx_v, flat_v, rows_v,
                 isems, gsems, osems):
    wid = lax.axis_index("s") * NC + lax.axis_index("c")
    bbase = wid * B_PER_W
    lane = lax.iota(jnp.int32, NLANES)

    def issue_idx(j, b):
        pltpu.async_copy(
            idx_hbm.at[:, pl.ds(bbase + j * CB, CB)], idx_v.at[b], isems[b])

    def wait_idx(b):
        pltpu.make_async_copy(
            idx_hbm.at[:, pl.ds(bbase, CB)], idx_v.at[b], isems[b]).wait()

    def compact(b):
        dst = flat_v.at[b]
        for s in range(S):
            x = idx_v[b, s, pl.ds(0, NLANES)]
            plsc.store_scatter(dst, [lane * S + s], x)

    def issue_gather(b):
        pltpu.async_copy(table_hbm.at[flat_v.at[b]], rows_v.at[b], gsems[b])

    def wait_gather(b):
        pltpu.make_async_copy(
            table_hbm.at[flat_v.at[b]], rows_v.at[b], gsems[b]).wait()

    def issue_out(j, b):
        pltpu.async_copy(
            rows_v.at[b],
            out_hbm.at[pl.ds((bbase + j * CB) * S, CROWS)], osems[b])

    def wait_out(b):
        pltpu.make_async_copy(
            rows_v.at[b], out_hbm.at[pl.ds(bbase * S, CROWS)],
            osems[b]).wait()

    def start_chunk(b):
        wait_idx(b)
        compact(b)
        issue_gather(b)

    def finalize(k, b, last):
        wait_gather(b)
        issue_out(k, b)
        if not last:
            issue_idx(jnp.minimum(k + NBUF, N_CHUNKS - 1), b)

    for b in range(NBUF):
        issue_idx(b, b)
    for b in range(NBUF):
        start_chunk(b)
        if b > 0:
            finalize(b - 1, b - 1, last=False)

    @pl.loop(1, N_GROUPS)
    def _group(g):
        j0 = g * NBUF
        for b in range(NBUF):
            j = j0 + b
            wait_out(b)
            start_chunk(b)
            pb = (b - 1) % NBUF
            finalize(j - 1, pb, last=False)

    last_b = (N_CHUNKS - 1) % NBUF
    finalize(N_CHUNKS - 1, last_b, last=True)
    for b in range(NBUF):
        wait_out(b)
    for b in range(NBUF):
        if b != last_b:
            wait_idx(b)


def kernel(word_idx, emb):
    idx_t = word_idx.T
    mesh = plsc.VectorSubcoreMesh(core_axis_name="c", subcore_axis_name="s")
    f = pl.kernel(
        _gather_body,
        out_type=jax.ShapeDtypeStruct((N_ROWS, D), jnp.float32),
        mesh=mesh,
        scratch_types=[
            pltpu.VMEM((NBUF, S, CB), jnp.int32),
            pltpu.VMEM((NBUF, CROWS), jnp.int32),
            pltpu.VMEM((NBUF, CROWS, D), jnp.float32),
            [pltpu.SemaphoreType.DMA] * NBUF,
            [pltpu.SemaphoreType.DMA] * NBUF,
            [pltpu.SemaphoreType.DMA] * NBUF,
        ],
        compiler_params=pltpu.CompilerParams(
            use_tc_tiling_on_sc=False, needs_layout_passes=False),
    )
    out = f(idx_t, emb)
    return out.reshape(B, S, D)


# speedup vs baseline: 1.0084x; 1.0032x over previous
import jax
import jax.numpy as jnp
from jax import lax
from jax.experimental import pallas as pl
from jax.experimental.pallas import tpu as pltpu
from jax.experimental.pallas import tpu_sc as plsc

B, S = 16384, 50
D = 64
N_ROWS = B * S
NC, NS = 2, 16
NW = NC * NS
B_PER_W = B // NW
NLANES = 16
NVB = B_PER_W // NLANES

CHUNK = 512
NBUF = 2
R_PER_W = N_ROWS // NW
N_CHUNKS = R_PER_W // CHUNK
N_GROUPS = N_CHUNKS // NBUF
assert R_PER_W % (CHUNK * NBUF) == 0


def _format_body(idx_hbm, flat_hbm, slab_v, flat_v, sems):
    wid = lax.axis_index("s") * NC + lax.axis_index("c")
    bbase = wid * B_PER_W
    lane = lax.iota(jnp.int32, NLANES)

    for s in range(S):
        pltpu.async_copy(
            idx_hbm.at[s, pl.ds(bbase, B_PER_W)], slab_v.at[s], sems[0])
    for s in range(S):
        pltpu.make_async_copy(
            idx_hbm.at[0, pl.ds(bbase, B_PER_W)], slab_v.at[s],
            sems[0]).wait()

    @pl.loop(0, NVB)
    def _vb(v):
        base = v * NLANES * S
        for s in range(S):
            x = slab_v[s, pl.ds(v * NLANES, NLANES)]
            plsc.store_scatter(flat_v, [lane * S + (base + s)], x)

    pltpu.async_copy(
        flat_v, flat_hbm.at[pl.ds(wid * R_PER_W, R_PER_W)], sems[1])
    pltpu.make_async_copy(
        flat_v, flat_hbm.at[pl.ds(wid * R_PER_W, R_PER_W)], sems[1]).wait()


def _gather_body(idx_hbm, table_hbm, out_hbm, idx_v, rows_v, isems, gsems, osems):
    wid = lax.axis_index("s") * NC + lax.axis_index("c")
    base = wid * R_PER_W

    def issue_idx(j, b):
        pltpu.async_copy(
            idx_hbm.at[pl.ds(base + j * CHUNK, CHUNK)], idx_v.at[b], isems[b])

    def wait_idx(b):
        pltpu.make_async_copy(
            idx_hbm.at[pl.ds(base, CHUNK)], idx_v.at[b], isems[b]).wait()

    def issue_gather(b):
        pltpu.async_copy(table_hbm.at[idx_v.at[b]], rows_v.at[b], gsems[b])

    def wait_gather(b):
        pltpu.make_async_copy(
            table_hbm.at[idx_v.at[b]], rows_v.at[b], gsems[b]).wait()

    def issue_out(j, b):
        pltpu.async_copy(
            rows_v.at[b], out_hbm.at[pl.ds(base + j * CHUNK, CHUNK)], osems[b])

    def wait_out(b):
        pltpu.make_async_copy(
            rows_v.at[b], out_hbm.at[pl.ds(base, CHUNK)], osems[b]).wait()

    def finalize(k, b, last):
        wait_gather(b)
        issue_out(k, b)
        if not last:
            issue_idx(jnp.minimum(k + NBUF, N_CHUNKS - 1), b)

    for b in range(NBUF):
        issue_idx(b, b)
    for b in range(NBUF):
        wait_idx(b)
        issue_gather(b)
        if b > 0:
            finalize(b - 1, b - 1, last=False)

    @pl.loop(1, N_GROUPS)
    def _group(g):
        j0 = g * NBUF
        for b in range(NBUF):
            j = j0 + b
            wait_idx(b)
            wait_out(b)
            issue_gather(b)
            pb = (b - 1) % NBUF
            finalize(j - 1, pb, last=False)

    last_b = (N_CHUNKS - 1) % NBUF
    finalize(N_CHUNKS - 1, last_b, last=True)
    for b in range(NBUF):
        wait_out(b)
    for b in range(NBUF):
        if b != last_b:
            wait_idx(b)


def kernel(word_idx, emb):
    mesh = plsc.VectorSubcoreMesh(core_axis_name="c", subcore_axis_name="s")

    fmt = pl.kernel(
        _format_body,
        out_type=jax.ShapeDtypeStruct((N_ROWS,), jnp.int32),
        mesh=mesh,
        scratch_types=[
            pltpu.VMEM((S, B_PER_W), jnp.int32),
            pltpu.VMEM((R_PER_W,), jnp.int32),
            [pltpu.SemaphoreType.DMA] * 2,
        ],
        compiler_params=pltpu.CompilerParams(
            use_tc_tiling_on_sc=True, needs_layout_passes=False),
    )
    flat_idx = fmt(word_idx.T)

    gather = pl.kernel(
        _gather_body,
        out_type=jax.ShapeDtypeStruct((N_ROWS, D), jnp.float32),
        mesh=mesh,
        scratch_types=[
            pltpu.VMEM((NBUF, CHUNK), jnp.int32),
            pltpu.VMEM((NBUF, CHUNK, D), jnp.float32),
            [pltpu.SemaphoreType.DMA] * NBUF,
            [pltpu.SemaphoreType.DMA] * NBUF,
            [pltpu.SemaphoreType.DMA] * NBUF,
        ],
        compiler_params=pltpu.CompilerParams(
            use_tc_tiling_on_sc=False, needs_layout_passes=False),
    )
    out = gather(flat_idx, emb)
    return out.reshape(B, S, D)


# speedup vs baseline: 1.0681x; 1.0593x over previous
import jax
import jax.numpy as jnp
from jax import lax
from jax.experimental import pallas as pl
from jax.experimental.pallas import tpu as pltpu
from jax.experimental.pallas import tpu_sc as plsc

B, S = 16384, 50
D = 64
N_ROWS = B * S
NC, NS = 2, 16
NW = NC * NS
B_PER_W = B // NW
NLANES = 16
NVB = B_PER_W // NLANES

CHUNK = 512
NBUF = 2
R_PER_W = N_ROWS // NW
N_CHUNKS = R_PER_W // CHUNK
N_GROUPS = N_CHUNKS // NBUF
assert R_PER_W % (CHUNK * NBUF) == 0


def _format_body(idx_hbm, flat_hbm, slab_v, flat_v, sems):
    wid = lax.axis_index("s") * NC + lax.axis_index("c")
    bbase = wid * B_PER_W
    lane = lax.iota(jnp.int32, NLANES)

    for s in range(S):
        pltpu.async_copy(
            idx_hbm.at[s, pl.ds(bbase, B_PER_W)], slab_v.at[s], sems[0])
    for s in range(S):
        pltpu.make_async_copy(
            idx_hbm.at[0, pl.ds(bbase, B_PER_W)], slab_v.at[s],
            sems[0]).wait()

    @pl.loop(0, NVB)
    def _vb(v):
        base = v * NLANES * S
        for s in range(S):
            x = slab_v[s, pl.ds(v * NLANES, NLANES)]
            plsc.store_scatter(flat_v, [lane * S + (base + s)], x + x)

    pltpu.async_copy(
        flat_v, flat_hbm.at[pl.ds(wid * R_PER_W, R_PER_W)], sems[1])
    pltpu.make_async_copy(
        flat_v, flat_hbm.at[pl.ds(wid * R_PER_W, R_PER_W)], sems[1]).wait()


def _gather_body(idx_hbm, table_hbm, out_hbm, idx_v, rows_v, isems, gsems, osems):
    wid = lax.axis_index("s") * NC + lax.axis_index("c")
    base = wid * R_PER_W

    def issue_idx(j, b):
        pltpu.async_copy(
            idx_hbm.at[pl.ds(base + j * CHUNK, CHUNK)], idx_v.at[b], isems[b])

    def wait_idx(b):
        pltpu.make_async_copy(
            idx_hbm.at[pl.ds(base, CHUNK)], idx_v.at[b], isems[b]).wait()

    def issue_gather(b):
        pltpu.async_copy(table_hbm.at[idx_v.at[b]], rows_v.at[b], gsems[b])

    def wait_gather(b):
        pltpu.make_async_copy(
            table_hbm.at[idx_v.at[b]], rows_v.at[b], gsems[b]).wait()

    def issue_out(j, b):
        pltpu.async_copy(
            rows_v.at[b], out_hbm.at[pl.ds(base + j * CHUNK, CHUNK)], osems[b])

    def wait_out(b):
        pltpu.make_async_copy(
            rows_v.at[b], out_hbm.at[pl.ds(base, CHUNK)], osems[b]).wait()

    def finalize(k, b, last):
        wait_gather(b)
        issue_out(k, b)
        if not last:
            issue_idx(jnp.minimum(k + NBUF, N_CHUNKS - 1), b)

    for b in range(NBUF):
        issue_idx(b, b)
    for b in range(NBUF):
        wait_idx(b)
        issue_gather(b)
        if b > 0:
            finalize(b - 1, b - 1, last=False)

    @pl.loop(1, N_GROUPS)
    def _group(g):
        j0 = g * NBUF
        for b in range(NBUF):
            j = j0 + b
            wait_idx(b)
            wait_out(b)
            issue_gather(b)
            pb = (b - 1) % NBUF
            finalize(j - 1, pb, last=False)

    last_b = (N_CHUNKS - 1) % NBUF
    finalize(N_CHUNKS - 1, last_b, last=True)
    for b in range(NBUF):
        wait_out(b)
    for b in range(NBUF):
        if b != last_b:
            wait_idx(b)


def kernel(word_idx, emb):
    mesh = plsc.VectorSubcoreMesh(core_axis_name="c", subcore_axis_name="s")

    fmt = pl.kernel(
        _format_body,
        out_type=jax.ShapeDtypeStruct((N_ROWS,), jnp.int32),
        mesh=mesh,
        scratch_types=[
            pltpu.VMEM((S, B_PER_W), jnp.int32),
            pltpu.VMEM((R_PER_W,), jnp.int32),
            [pltpu.SemaphoreType.DMA] * 2,
        ],
        compiler_params=pltpu.CompilerParams(
            use_tc_tiling_on_sc=True, needs_layout_passes=False),
    )
    flat_idx = fmt(word_idx.T)

    gather = pl.kernel(
        _gather_body,
        out_type=jax.ShapeDtypeStruct((N_ROWS, D), jnp.float32),
        mesh=mesh,
        scratch_types=[
            pltpu.VMEM((NBUF, CHUNK), jnp.int32),
            pltpu.VMEM((NBUF, CHUNK, D), jnp.float32),
            [pltpu.SemaphoreType.DMA] * NBUF,
            [pltpu.SemaphoreType.DMA] * NBUF,
            [pltpu.SemaphoreType.DMA] * NBUF,
        ],
        compiler_params=pltpu.CompilerParams(
            use_tc_tiling_on_sc=False, needs_layout_passes=False),
    )
    emb2 = jnp.pad(emb, ((0, 0), (0, D))).reshape(2 * emb.shape[0], D)
    out = gather(flat_idx, emb2)
    return out.reshape(B, S, D)
